# baseline (speedup 1.0000x reference)
import dataclasses
import functools

import jax
import jax.numpy as jnp
from jax import lax
from jax.experimental import pallas as pl
from jax.experimental.pallas import tpu as pltpu
from jax.experimental.pallas import tpu_sc as plsc

N = 10000
E = 320000
IN = 128
H1 = 128
H2 = 64
EPS = 1e-5

NC = 2
NS = 16
NW = NC * NS
CHUNK = 128
NSLAB = 80
EPT = NSLAB * CHUNK
EP = NW * EPT
NPAD = NSLAB * CHUNK

_mesh = plsc.VectorSubcoreMesh(core_axis_name="c", subcore_axis_name="s")

_sc_params = pltpu.CompilerParams()
if "needs_layout_passes" in pltpu.CompilerParams.__dataclass_fields__:
    _sc_params = dataclasses.replace(_sc_params, needs_layout_passes=False)


def _deg_sc(dst_p3, w_p3):
    @functools.partial(
        pl.kernel,
        out_type=jax.ShapeDtypeStruct((NC, NPAD, 16), jnp.float32),
        mesh=_mesh,
        compiler_params=_sc_params,
        scratch_types=[
            pltpu.VMEM((NSLAB, CHUNK), jnp.int32),
            pltpu.VMEM((NSLAB, CHUNK), jnp.float32),
            pltpu.VMEM((CHUNK, 16), jnp.float32),
            pltpu.VMEM_SHARED((NPAD, 16), jnp.float32),
        ],
    )
    def deg_kernel(dst_hbm, w_hbm, out_hbm, dstb, wb, rows, acc):
        cid = lax.axis_index("c")
        sid = lax.axis_index("s")
        wid = cid * NS + sid
        pltpu.sync_copy(dst_hbm.at[wid], dstb)
        pltpu.sync_copy(w_hbm.at[wid], wb)

        zeros16 = jnp.zeros((16,), jnp.float32)

        @pl.loop(0, CHUNK)
        def _(r):
            rows[r, pl.ds(0, 16)] = zeros16

        @pl.loop(sid, NSLAB, step=NS)
        def _(s):
            pltpu.sync_copy(rows, acc.at[pl.ds(s * CHUNK, CHUNK)])

        plsc.subcore_barrier()

        ones16 = jnp.ones((16,), jnp.float32)

        @pl.loop(0, NSLAB)
        def _(g):
            @pl.loop(0, CHUNK, step=16)
            def _(i):
                wv = wb[g, pl.ds(i, 16)]
                for k in range(16):
                    rows[i + k, pl.ds(0, 16)] = ones16 * wv[k]

            pltpu.sync_copy(rows, acc.at[dstb.at[g]], add=True)

        plsc.subcore_barrier()

        @pl.loop(sid, NSLAB, step=NS)
        def _(s):
            pltpu.sync_copy(acc.at[pl.ds(s * CHUNK, CHUNK)],
                            out_hbm.at[cid, pl.ds(s * CHUNK, CHUNK)])

    return deg_kernel(dst_p3, w_p3)


def _agg_sc(src_p3, dst_p3, w_p3, h, D, DO, PHASES):
    CH = CHUNK
    NCH = EPT // CH
    BCH = NCH // PHASES
    SLABS = NPAD // CH
    inplace = (DO == D)

    scratch = [
        pltpu.VMEM((BCH, CH), jnp.int32),
        pltpu.VMEM((BCH, CH), jnp.int32),
        pltpu.VMEM((BCH, CH), jnp.float32),
        pltpu.VMEM((CH, D), jnp.float32),
        pltpu.VMEM((CH, D), jnp.float32),
        pltpu.VMEM_SHARED((NPAD, DO), jnp.float32),
        pltpu.SemaphoreType.DMA,
        pltpu.SemaphoreType.DMA,
    ]
    if not inplace:
        scratch[5:5] = [pltpu.VMEM((CH, DO), jnp.float32),
                        pltpu.VMEM((CH, DO), jnp.float32)]

    @functools.partial(
        pl.kernel,
        out_type=jax.ShapeDtypeStruct((NC, NPAD, DO), jnp.float32),
        mesh=_mesh,
        compiler_params=_sc_params,
        scratch_types=scratch,
    )
    def agg_kernel(src_hbm, dst_hbm, w_hbm, h_hbm, out_hbm,
                   srcb, dstb, wb, rows0, rows1, *rest):
        if inplace:
            acc, sem0, sem1 = rest
            sc0, sc1 = rows0, rows1
        else:
            sc0, sc1, acc, sem0, sem1 = rest
        cid = lax.axis_index("c")
        sid = lax.axis_index("s")
        wid = cid * NS + sid

        zeros16 = jnp.zeros((16,), jnp.float32)

        @pl.loop(0, CH)
        def _(r):
            for j in range(DO // 16):
                sc0[r, pl.ds(j * 16, 16)] = zeros16

        @pl.loop(sid, SLABS, step=NS)
        def _(s):
            pltpu.sync_copy(sc0, acc.at[pl.ds(s * CH, CH)])

        plsc.subcore_barrier()

        def start_gather(g, buf, sem):
            pltpu.make_async_copy(h_hbm.at[srcb.at[g]], buf, sem).start()

        def wait_gather(g, buf, sem):
            pltpu.make_async_copy(h_hbm.at[srcb.at[g]], buf, sem).wait()

        def process(gg, buf, sbuf):
            pltpu.sync_copy(h_hbm.at[srcb.at[gg]], buf)

            @pl.loop(0, CH, step=16)
            def _(i):
                wv = wb[gg, pl.ds(i, 16)]
                for k in range(16):
                    ws = wv[k]
                    for j in range(DO // 16):
                        sbuf[i + k, pl.ds(j * 16, 16)] = (
                            buf[i + k, pl.ds(j * 16, 16)] * ws)

            pltpu.sync_copy(sbuf, acc.at[dstb.at[gg]], add=True)

        for p in range(PHASES):
            pltpu.sync_copy(src_hbm.at[wid, pl.ds(p * BCH, BCH)], srcb)
            pltpu.sync_copy(dst_hbm.at[wid, pl.ds(p * BCH, BCH)], dstb)
            pltpu.sync_copy(w_hbm.at[wid, pl.ds(p * BCH, BCH)], wb)

            @pl.loop(0, BCH)
            def _(g):
                process(g, rows0, sc0)

        plsc.subcore_barrier()

        @pl.loop(sid, SLABS, step=NS)
        def _(s):
            pltpu.sync_copy(acc.at[pl.ds(s * CH, CH)],
                            out_hbm.at[cid, pl.ds(s * CH, CH)])

    return agg_kernel(src_p3, dst_p3, w_p3, h)


def _tc1(x, W1, degp):
    def body(x_ref, w1_ref, degp_ref, dinv_ref, h1s_ref):
        deg = 1.0 + degp_ref[0, :, 0:1] + degp_ref[1, :, 0:1]
        dinv = lax.rsqrt(deg)
        dinv_ref[...] = dinv
        h1 = lax.dot_general(x_ref[...], w1_ref[...], (((1,), (1,)), ((), ())),
                             preferred_element_type=jnp.float32)
        h1s_ref[...] = h1 * dinv[:N]

    return pl.pallas_call(
        body,
        out_shape=(jax.ShapeDtypeStruct((NPAD, 1), jnp.float32),
                   jax.ShapeDtypeStruct((N, H1), jnp.float32)),
    )(x, W1, degp)


def _tc2(p, h1s, dinv, b1, gamma1, beta1, W2):
    def body(p_ref, h1s_ref, dinv_ref, b1_ref, g1_ref, be1_ref, w2_ref,
             h2s_ref):
        dv = dinv_ref[pl.ds(0, N), :]
        agg = p_ref[0, :N, :] + p_ref[1, :N, :] + h1s_ref[...]
        out1 = dv * agg + b1_ref[...]
        mean = jnp.mean(out1, axis=0, keepdims=True)
        var = jnp.mean((out1 - mean) ** 2, axis=0, keepdims=True)
        hbn = (out1 - mean) / jnp.sqrt(var + EPS) * g1_ref[...] + be1_ref[...]
        hr = jnp.maximum(hbn, 0.0)
        h2 = lax.dot_general(hr, w2_ref[...], (((1,), (1,)), ((), ())),
                             preferred_element_type=jnp.float32)
        h2s = h2 * dv
        h2s_ref[...] = jnp.concatenate(
            [h2s, jnp.zeros((N, H1 - H2), jnp.float32)], axis=1)

    return pl.pallas_call(
        body,
        out_shape=jax.ShapeDtypeStruct((N, H1), jnp.float32),
    )(p, h1s, dinv, b1, gamma1, beta1, W2)


def _tc3(q, h2s, dinv, b2, gamma2, beta2, Wlin, blin):
    def body(q_ref, h2s_ref, dinv_ref, b2_ref, g2_ref, be2_ref, wl_ref,
             bl_ref, y_ref):
        dv = dinv_ref[pl.ds(0, N), :]
        agg = (q_ref[0, :N, :H2] + q_ref[1, :N, :H2] + h2s_ref[:, :H2])
        out2 = dv * agg + b2_ref[...]
        mean = jnp.mean(out2, axis=0, keepdims=True)
        var = jnp.mean((out2 - mean) ** 2, axis=0, keepdims=True)
        hbn = (out2 - mean) / jnp.sqrt(var + EPS) * g2_ref[...] + be2_ref[...]
        hr = jnp.maximum(hbn, 0.0)
        y = lax.dot_general(hr, wl_ref[...], (((1,), (1,)), ((), ())),
                            preferred_element_type=jnp.float32)
        y_ref[...] = y + bl_ref[0, 0]

    return pl.pallas_call(
        body,
        out_shape=jax.ShapeDtypeStruct((N, H1), jnp.float32),
    )(q, h2s, dinv, b2, gamma2, beta2, Wlin, blin)


def kernel(x, edge_index, edge_weight, W1, b1, gamma1, beta1,
           W2, b2, gamma2, beta2, Wlin, blin):
    src = edge_index[0]
    dst = edge_index[1]
    pad = EP - E
    shp = (NW, EPT // CHUNK, CHUNK)
    src_p3 = jnp.concatenate([src, jnp.zeros((pad,), jnp.int32)]).reshape(shp)
    dst_p3 = jnp.concatenate([dst, jnp.zeros((pad,), jnp.int32)]).reshape(shp)
    w_p3 = jnp.concatenate(
        [edge_weight, jnp.zeros((pad,), jnp.float32)]).reshape(shp)

    degp = _deg_sc(dst_p3, w_p3)
    dinv, h1s = _tc1(x, W1, degp)
    p1 = _agg_sc(src_p3, dst_p3, w_p3, h1s, H1, H1, 2)
    h2s = _tc2(p1, h1s, dinv, b1.reshape(1, H1), gamma1.reshape(1, H1),
               beta1.reshape(1, H1), W2)
    q2 = _agg_sc(src_p3, dst_p3, w_p3, h2s, H1, H1, 2)
    wl_b = jnp.broadcast_to(Wlin, (H1, H2))
    y = _tc3(q2, h2s, dinv, b2.reshape(1, H2), gamma2.reshape(1, H2),
             beta2.reshape(1, H2), wl_b, blin.reshape(1, 1))
    return y[:, 0]


# speedup vs baseline: 2.2306x; 2.2306x over previous
import dataclasses
import functools

import jax
import jax.numpy as jnp
from jax import lax
from jax.experimental import pallas as pl
from jax.experimental.pallas import tpu as pltpu
from jax.experimental.pallas import tpu_sc as plsc

N = 10000
E = 320000
IN = 128
H1 = 128
H2 = 64
EPS = 1e-5

NC = 2
NS = 16
NW = NC * NS
CHUNK = 128
NSLAB = 80
EPT = NSLAB * CHUNK
EP = NW * EPT
NPAD = NSLAB * CHUNK

_mesh = plsc.VectorSubcoreMesh(core_axis_name="c", subcore_axis_name="s")

_sc_params = pltpu.CompilerParams()
if "needs_layout_passes" in pltpu.CompilerParams.__dataclass_fields__:
    _sc_params = dataclasses.replace(_sc_params, needs_layout_passes=False)


def _deg_sc(dst_p3, w_p3):
    @functools.partial(
        pl.kernel,
        out_type=jax.ShapeDtypeStruct((NC, NPAD, 16), jnp.float32),
        mesh=_mesh,
        compiler_params=_sc_params,
        scratch_types=[
            pltpu.VMEM((NSLAB, CHUNK), jnp.int32),
            pltpu.VMEM((NSLAB, CHUNK), jnp.float32),
            pltpu.VMEM((CHUNK, 16), jnp.float32),
            pltpu.VMEM_SHARED((NPAD, 16), jnp.float32),
        ],
    )
    def deg_kernel(dst_hbm, w_hbm, out_hbm, dstb, wb, rows, acc):
        cid = lax.axis_index("c")
        sid = lax.axis_index("s")
        wid = cid * NS + sid
        pltpu.sync_copy(dst_hbm.at[wid], dstb)
        pltpu.sync_copy(w_hbm.at[wid], wb)

        zeros16 = jnp.zeros((16,), jnp.float32)

        @pl.loop(0, CHUNK)
        def _(r):
            rows[r, pl.ds(0, 16)] = zeros16

        @pl.loop(sid, NSLAB, step=NS)
        def _(s):
            pltpu.sync_copy(rows, acc.at[pl.ds(s * CHUNK, CHUNK)])

        plsc.subcore_barrier()

        ones16 = jnp.ones((16,), jnp.float32)

        @pl.loop(0, NSLAB)
        def _(g):
            @pl.loop(0, CHUNK, step=16)
            def _(i):
                wv = wb[g, pl.ds(i, 16)]
                for k in range(16):
                    rows[i + k, pl.ds(0, 16)] = ones16 * wv[k]

            pltpu.sync_copy(rows, acc.at[dstb.at[g]], add=True)

        plsc.subcore_barrier()

        @pl.loop(sid, NSLAB, step=NS)
        def _(s):
            pltpu.sync_copy(acc.at[pl.ds(s * CHUNK, CHUNK)],
                            out_hbm.at[cid, pl.ds(s * CHUNK, CHUNK)])

    return deg_kernel(dst_p3, w_p3)


def _agg_sc(src_p3, dst_p3, w_p3, h, D, DO, PHASES):
    CH = CHUNK
    NCH = EPT // CH
    BCH = NCH // PHASES
    SLABS = NPAD // CH
    inplace = (DO == D)

    scratch = [
        pltpu.VMEM((BCH, CH), jnp.int32),
        pltpu.VMEM((BCH, CH), jnp.int32),
        pltpu.VMEM((BCH, CH), jnp.float32),
        pltpu.VMEM((CH, D), jnp.float32),
        pltpu.VMEM((CH, D), jnp.float32),
        pltpu.VMEM_SHARED((NPAD, DO), jnp.float32),
        pltpu.SemaphoreType.DMA,
        pltpu.SemaphoreType.DMA,
    ]
    if not inplace:
        scratch[5:5] = [pltpu.VMEM((CH, DO), jnp.float32),
                        pltpu.VMEM((CH, DO), jnp.float32)]

    @functools.partial(
        pl.kernel,
        out_type=jax.ShapeDtypeStruct((NC, NPAD, DO), jnp.float32),
        mesh=_mesh,
        compiler_params=_sc_params,
        scratch_types=scratch,
    )
    def agg_kernel(src_hbm, dst_hbm, w_hbm, h_hbm, out_hbm,
                   srcb, dstb, wb, rows0, rows1, *rest):
        if inplace:
            acc, sem0, sem1 = rest
            sc0, sc1 = rows0, rows1
        else:
            sc0, sc1, acc, sem0, sem1 = rest
        cid = lax.axis_index("c")
        sid = lax.axis_index("s")
        wid = cid * NS + sid

        zeros16 = jnp.zeros((16,), jnp.float32)

        @pl.loop(0, CH)
        def _(r):
            for j in range(DO // 16):
                sc0[r, pl.ds(j * 16, 16)] = zeros16

        @pl.loop(sid, SLABS, step=NS)
        def _(s):
            pltpu.sync_copy(sc0, acc.at[pl.ds(s * CH, CH)])

        plsc.subcore_barrier()

        def start_gather(g, buf, sem):
            pltpu.make_async_copy(h_hbm.at[srcb.at[g]], buf, sem).start()

        def wait_gather(g, buf, sem):
            pltpu.make_async_copy(h_hbm.at[srcb.at[g]], buf, sem).wait()

        def process(gg, buf, sbuf):
            pltpu.sync_copy(h_hbm.at[srcb.at[gg]], buf)

            @pl.loop(0, CH, step=16)
            def _(i):
                wv = wb[gg, pl.ds(i, 16)]
                for k in range(16):
                    ws = wv[k]
                    for j in range(DO // 16):
                        sbuf[i + k, pl.ds(j * 16, 16)] = (
                            buf[i + k, pl.ds(j * 16, 16)] * ws)

            pltpu.sync_copy(sbuf, acc.at[dstb.at[gg]], add=True)

        for p in range(PHASES):
            pltpu.sync_copy(src_hbm.at[wid, pl.ds(p * BCH, BCH)], srcb)
            pltpu.sync_copy(dst_hbm.at[wid, pl.ds(p * BCH, BCH)], dstb)
            pltpu.sync_copy(w_hbm.at[wid, pl.ds(p * BCH, BCH)], wb)

            @pl.loop(0, BCH)
            def _(g):
                process(g, rows0, sc0)

        plsc.subcore_barrier()

        @pl.loop(sid, SLABS, step=NS)
        def _(s):
            pltpu.sync_copy(acc.at[pl.ds(s * CH, CH)],
                            out_hbm.at[cid, pl.ds(s * CH, CH)])

    return agg_kernel(src_p3, dst_p3, w_p3, h)


def _tc1(x, W1, degp):
    def body(x_ref, w1_ref, degp_ref, dinv_ref, h1s_ref):
        deg = 1.0 + degp_ref[0, :, 0:1] + degp_ref[1, :, 0:1]
        dinv = lax.rsqrt(deg)
        dinv_ref[...] = dinv
        h1 = lax.dot_general(x_ref[...], w1_ref[...], (((1,), (1,)), ((), ())),
                             preferred_element_type=jnp.float32)
        h1s_ref[...] = h1 * dinv[:N]

    return pl.pallas_call(
        body,
        out_shape=(jax.ShapeDtypeStruct((NPAD, 1), jnp.float32),
                   jax.ShapeDtypeStruct((N, H1), jnp.float32)),
    )(x, W1, degp)


def _tc2(p, h1s, dinv, b1, gamma1, beta1, W2):
    def body(p_ref, h1s_ref, dinv_ref, b1_ref, g1_ref, be1_ref, w2_ref,
             h2s_ref):
        dv = dinv_ref[pl.ds(0, N), :]
        agg = p_ref[0, :N, :] + p_ref[1, :N, :] + h1s_ref[...]
        out1 = dv * agg + b1_ref[...]
        mean = jnp.mean(out1, axis=0, keepdims=True)
        var = jnp.mean((out1 - mean) ** 2, axis=0, keepdims=True)
        hbn = (out1 - mean) / jnp.sqrt(var + EPS) * g1_ref[...] + be1_ref[...]
        hr = jnp.maximum(hbn, 0.0)
        h2 = lax.dot_general(hr, w2_ref[...], (((1,), (1,)), ((), ())),
                             preferred_element_type=jnp.float32)
        h2s = h2 * dv
        h2s_ref[...] = jnp.concatenate(
            [h2s, jnp.zeros((N, H1 - H2), jnp.float32)], axis=1)

    return pl.pallas_call(
        body,
        out_shape=jax.ShapeDtypeStruct((N, H1), jnp.float32),
    )(p, h1s, dinv, b1, gamma1, beta1, W2)


def _tc3(q, h2s, dinv, b2, gamma2, beta2, Wlin, blin):
    def body(q_ref, h2s_ref, dinv_ref, b2_ref, g2_ref, be2_ref, wl_ref,
             bl_ref, y_ref):
        dv = dinv_ref[pl.ds(0, N), :]
        agg = (q_ref[0, :N, :H2] + q_ref[1, :N, :H2] + h2s_ref[:, :H2])
        out2 = dv * agg + b2_ref[...]
        mean = jnp.mean(out2, axis=0, keepdims=True)
        var = jnp.mean((out2 - mean) ** 2, axis=0, keepdims=True)
        hbn = (out2 - mean) / jnp.sqrt(var + EPS) * g2_ref[...] + be2_ref[...]
        hr = jnp.maximum(hbn, 0.0)
        y = lax.dot_general(hr, wl_ref[...], (((1,), (1,)), ((), ())),
                            preferred_element_type=jnp.float32)
        y_ref[...] = y + bl_ref[0, 0]

    return pl.pallas_call(
        body,
        out_shape=jax.ShapeDtypeStruct((N, H1), jnp.float32),
    )(q, h2s, dinv, b2, gamma2, beta2, Wlin, blin)


def kernel(x, edge_index, edge_weight, W1, b1, gamma1, beta1,
           W2, b2, gamma2, beta2, Wlin, blin):
    src = edge_index[0]
    dst = edge_index[1]
    pad = EP - E
    shp = (NW, EPT // CHUNK, CHUNK)
    pad_idx = jnp.arange(pad, dtype=jnp.int32) % N
    src_p3 = jnp.concatenate([src, pad_idx]).reshape(shp)
    dst_p3 = jnp.concatenate([dst, pad_idx]).reshape(shp)
    w_p3 = jnp.concatenate(
        [edge_weight, jnp.zeros((pad,), jnp.float32)]).reshape(shp)

    degp = _deg_sc(dst_p3, w_p3)
    dinv, h1s = _tc1(x, W1, degp)
    p1 = _agg_sc(src_p3, dst_p3, w_p3, h1s, H1, H1, 2)
    h2s = _tc2(p1, h1s, dinv, b1.reshape(1, H1), gamma1.reshape(1, H1),
               beta1.reshape(1, H1), W2)
    q2 = _agg_sc(src_p3, dst_p3, w_p3, h2s, H1, H1, 2)
    wl_b = jnp.broadcast_to(Wlin, (H1, H2))
    y = _tc3(q2, h2s, dinv, b2.reshape(1, H2), gamma2.reshape(1, H2),
             beta2.reshape(1, H2), wl_b, blin.reshape(1, 1))
    return y[:, 0]


# speedup vs baseline: 3.2633x; 1.4630x over previous
import dataclasses
import functools

import jax
import jax.numpy as jnp
from jax import lax
from jax.experimental import pallas as pl
from jax.experimental.pallas import tpu as pltpu
from jax.experimental.pallas import tpu_sc as plsc

N = 10000
E = 320000
IN = 128
H1 = 128
H2 = 64
EPS = 1e-5

NC = 2
NS = 16
NW = NC * NS
CHUNK = 128
NSLAB = 80
EPT = NSLAB * CHUNK
EP = NW * EPT
NPAD = NSLAB * CHUNK

_mesh = plsc.VectorSubcoreMesh(core_axis_name="c", subcore_axis_name="s")

_sc_params = pltpu.CompilerParams()
if "needs_layout_passes" in pltpu.CompilerParams.__dataclass_fields__:
    _sc_params = dataclasses.replace(_sc_params, needs_layout_passes=False)


def _deg_sc(dst_p3, w_p3):
    @functools.partial(
        pl.kernel,
        out_type=jax.ShapeDtypeStruct((NC, NPAD, 16), jnp.float32),
        mesh=_mesh,
        compiler_params=_sc_params,
        scratch_types=[
            pltpu.VMEM((NSLAB, CHUNK), jnp.int32),
            pltpu.VMEM((NSLAB, CHUNK), jnp.float32),
            pltpu.VMEM((CHUNK, 16), jnp.float32),
            pltpu.VMEM_SHARED((NPAD, 16), jnp.float32),
        ],
    )
    def deg_kernel(dst_hbm, w_hbm, out_hbm, dstb, wb, rows, acc):
        cid = lax.axis_index("c")
        sid = lax.axis_index("s")
        wid = cid * NS + sid
        pltpu.sync_copy(dst_hbm.at[wid], dstb)
        pltpu.sync_copy(w_hbm.at[wid], wb)

        zeros16 = jnp.zeros((16,), jnp.float32)

        @pl.loop(0, CHUNK)
        def _(r):
            rows[r, pl.ds(0, 16)] = zeros16

        @pl.loop(sid, NSLAB, step=NS)
        def _(s):
            pltpu.sync_copy(rows, acc.at[pl.ds(s * CHUNK, CHUNK)])

        plsc.subcore_barrier()

        ones16 = jnp.ones((16,), jnp.float32)

        @pl.loop(0, NSLAB)
        def _(g):
            @pl.loop(0, CHUNK, step=16)
            def _(i):
                wv = wb[g, pl.ds(i, 16)]
                for k in range(16):
                    rows[i + k, pl.ds(0, 16)] = ones16 * wv[k]

            pltpu.sync_copy(rows, acc.at[dstb.at[g]], add=True)

        plsc.subcore_barrier()

        @pl.loop(sid, NSLAB, step=NS)
        def _(s):
            pltpu.sync_copy(acc.at[pl.ds(s * CHUNK, CHUNK)],
                            out_hbm.at[cid, pl.ds(s * CHUNK, CHUNK)])

    return deg_kernel(dst_p3, w_p3)


def _agg_sc(src_p3, dst_p3, w_p3, h, D, DO, PHASES):
    CH = CHUNK
    NCH = EPT // CH
    BCH = NCH // PHASES
    SLABS = NPAD // CH
    inplace = (DO == D)

    scratch = [
        pltpu.VMEM((BCH, CH), jnp.int32),
        pltpu.VMEM((BCH, CH), jnp.int32),
        pltpu.VMEM((BCH, CH), jnp.float32),
        pltpu.VMEM((CH, D), jnp.float32),
        pltpu.VMEM((CH, D), jnp.float32),
        pltpu.VMEM_SHARED((NPAD, DO), jnp.float32),
        pltpu.SemaphoreType.DMA,
        pltpu.SemaphoreType.DMA,
    ]
    if not inplace:
        scratch[5:5] = [pltpu.VMEM((CH, DO), jnp.float32),
                        pltpu.VMEM((CH, DO), jnp.float32)]

    @functools.partial(
        pl.kernel,
        out_type=jax.ShapeDtypeStruct((NC, NPAD, DO), jnp.float32),
        mesh=_mesh,
        compiler_params=_sc_params,
        scratch_types=scratch,
    )
    def agg_kernel(src_hbm, dst_hbm, w_hbm, h_hbm, out_hbm,
                   srcb, dstb, wb, rows0, rows1, *rest):
        if inplace:
            acc, sem0, sem1 = rest
            sc0, sc1 = rows0, rows1
        else:
            sc0, sc1, acc, sem0, sem1 = rest
        cid = lax.axis_index("c")
        sid = lax.axis_index("s")
        wid = cid * NS + sid

        zeros16 = jnp.zeros((16,), jnp.float32)

        @pl.loop(0, CH)
        def _(r):
            for j in range(DO // 16):
                sc0[r, pl.ds(j * 16, 16)] = zeros16

        @pl.loop(sid, SLABS, step=NS)
        def _(s):
            pltpu.sync_copy(sc0, acc.at[pl.ds(s * CH, CH)])

        plsc.subcore_barrier()

        def start_gather(g, buf, sem):
            pltpu.make_async_copy(h_hbm.at[srcb.at[g]], buf, sem).start()

        def wait_gather(g, buf, sem):
            pltpu.make_async_copy(h_hbm.at[srcb.at[g]], buf, sem).wait()

        def process(gg, buf, sbuf, sem):
            wait_gather(gg, buf, sem)

            @pl.loop(0, CH, step=16)
            def _(i):
                wv = wb[gg, pl.ds(i, 16)]
                for k in range(16):
                    ws = wv[k]
                    for j in range(DO // 16):
                        sbuf[i + k, pl.ds(j * 16, 16)] = (
                            buf[i + k, pl.ds(j * 16, 16)] * ws)

            pltpu.sync_copy(sbuf, acc.at[dstb.at[gg]], add=True)

        for p in range(PHASES):
            pltpu.sync_copy(src_hbm.at[wid, pl.ds(p * BCH, BCH)], srcb)
            pltpu.sync_copy(dst_hbm.at[wid, pl.ds(p * BCH, BCH)], dstb)
            pltpu.sync_copy(w_hbm.at[wid, pl.ds(p * BCH, BCH)], wb)

            start_gather(0, rows0, sem0)
            start_gather(1, rows1, sem1)

            @pl.loop(0, BCH - 2, step=2)
            def _(g):
                for buf, sbuf, sem, off in ((rows0, sc0, sem0, 0),
                                            (rows1, sc1, sem1, 1)):
                    gg = g + off
                    process(gg, buf, sbuf, sem)
                    start_gather(gg + 2, buf, sem)

            process(BCH - 2, rows0, sc0, sem0)
            process(BCH - 1, rows1, sc1, sem1)

        plsc.subcore_barrier()

        @pl.loop(sid, SLABS, step=NS)
        def _(s):
            pltpu.sync_copy(acc.at[pl.ds(s * CH, CH)],
                            out_hbm.at[cid, pl.ds(s * CH, CH)])

    return agg_kernel(src_p3, dst_p3, w_p3, h)


def _tc1(x, W1, degp):
    def body(x_ref, w1_ref, degp_ref, dinv_ref, h1s_ref):
        deg = 1.0 + degp_ref[0, :, 0:1] + degp_ref[1, :, 0:1]
        dinv = lax.rsqrt(deg)
        dinv_ref[...] = dinv
        h1 = lax.dot_general(x_ref[...], w1_ref[...], (((1,), (1,)), ((), ())),
                             preferred_element_type=jnp.float32)
        h1s_ref[...] = h1 * dinv[:N]

    return pl.pallas_call(
        body,
        out_shape=(jax.ShapeDtypeStruct((NPAD, 1), jnp.float32),
                   jax.ShapeDtypeStruct((N, H1), jnp.float32)),
    )(x, W1, degp)


def _tc2(p, h1s, dinv, b1, gamma1, beta1, W2):
    def body(p_ref, h1s_ref, dinv_ref, b1_ref, g1_ref, be1_ref, w2_ref,
             h2s_ref):
        dv = dinv_ref[pl.ds(0, N), :]
        agg = p_ref[0, :N, :] + p_ref[1, :N, :] + h1s_ref[...]
        out1 = dv * agg + b1_ref[...]
        mean = jnp.mean(out1, axis=0, keepdims=True)
        var = jnp.mean((out1 - mean) ** 2, axis=0, keepdims=True)
        hbn = (out1 - mean) / jnp.sqrt(var + EPS) * g1_ref[...] + be1_ref[...]
        hr = jnp.maximum(hbn, 0.0)
        h2 = lax.dot_general(hr, w2_ref[...], (((1,), (1,)), ((), ())),
                             preferred_element_type=jnp.float32)
        h2s = h2 * dv
        h2s_ref[...] = jnp.concatenate(
            [h2s, jnp.zeros((N, H1 - H2), jnp.float32)], axis=1)

    return pl.pallas_call(
        body,
        out_shape=jax.ShapeDtypeStruct((N, H1), jnp.float32),
    )(p, h1s, dinv, b1, gamma1, beta1, W2)


def _tc3(q, h2s, dinv, b2, gamma2, beta2, Wlin, blin):
    def body(q_ref, h2s_ref, dinv_ref, b2_ref, g2_ref, be2_ref, wl_ref,
             bl_ref, y_ref):
        dv = dinv_ref[pl.ds(0, N), :]
        agg = (q_ref[0, :N, :H2] + q_ref[1, :N, :H2] + h2s_ref[:, :H2])
        out2 = dv * agg + b2_ref[...]
        mean = jnp.mean(out2, axis=0, keepdims=True)
        var = jnp.mean((out2 - mean) ** 2, axis=0, keepdims=True)
        hbn = (out2 - mean) / jnp.sqrt(var + EPS) * g2_ref[...] + be2_ref[...]
        hr = jnp.maximum(hbn, 0.0)
        y = lax.dot_general(hr, wl_ref[...], (((1,), (1,)), ((), ())),
                            preferred_element_type=jnp.float32)
        y_ref[...] = y + bl_ref[0, 0]

    return pl.pallas_call(
        body,
        out_shape=jax.ShapeDtypeStruct((N, H1), jnp.float32),
    )(q, h2s, dinv, b2, gamma2, beta2, Wlin, blin)


def kernel(x, edge_index, edge_weight, W1, b1, gamma1, beta1,
           W2, b2, gamma2, beta2, Wlin, blin):
    src = edge_index[0]
    dst = edge_index[1]
    pad = EP - E
    shp = (NW, EPT // CHUNK, CHUNK)
    pad_idx = jnp.arange(pad, dtype=jnp.int32) % N
    src_p3 = jnp.concatenate([src, pad_idx]).reshape(shp)
    dst_p3 = jnp.concatenate([dst, pad_idx]).reshape(shp)
    w_p3 = jnp.concatenate(
        [edge_weight, jnp.zeros((pad,), jnp.float32)]).reshape(shp)

    degp = _deg_sc(dst_p3, w_p3)
    dinv, h1s = _tc1(x, W1, degp)
    p1 = _agg_sc(src_p3, dst_p3, w_p3, h1s, H1, H1, 2)
    h2s = _tc2(p1, h1s, dinv, b1.reshape(1, H1), gamma1.reshape(1, H1),
               beta1.reshape(1, H1), W2)
    q2 = _agg_sc(src_p3, dst_p3, w_p3, h2s, H1, H1, 2)
    wl_b = jnp.broadcast_to(Wlin, (H1, H2))
    y = _tc3(q2, h2s, dinv, b2.reshape(1, H2), gamma2.reshape(1, H2),
             beta2.reshape(1, H2), wl_b, blin.reshape(1, 1))
    return y[:, 0]


# speedup vs baseline: 3.2714x; 1.0025x over previous
import dataclasses
import functools

import jax
import jax.numpy as jnp
from jax import lax
from jax.experimental import pallas as pl
from jax.experimental.pallas import tpu as pltpu
from jax.experimental.pallas import tpu_sc as plsc

N = 10000
E = 320000
IN = 128
H1 = 128
H2 = 64
EPS = 1e-5

NC = 2
NS = 16
NW = NC * NS
CHUNK = 128
NSLAB = 80
EPT = NSLAB * CHUNK
EP = NW * EPT
NPAD = NSLAB * CHUNK

_mesh = plsc.VectorSubcoreMesh(core_axis_name="c", subcore_axis_name="s")

_sc_params = pltpu.CompilerParams()
if "needs_layout_passes" in pltpu.CompilerParams.__dataclass_fields__:
    _sc_params = dataclasses.replace(_sc_params, needs_layout_passes=False)


def _deg_sc(dst_p3, w_p3):
    @functools.partial(
        pl.kernel,
        out_type=jax.ShapeDtypeStruct((NC, NPAD, 16), jnp.float32),
        mesh=_mesh,
        compiler_params=_sc_params,
        scratch_types=[
            pltpu.VMEM((NSLAB, CHUNK), jnp.int32),
            pltpu.VMEM((NSLAB, CHUNK), jnp.float32),
            pltpu.VMEM((CHUNK, 16), jnp.float32),
            pltpu.VMEM_SHARED((NPAD, 16), jnp.float32),
        ],
    )
    def deg_kernel(dst_hbm, w_hbm, out_hbm, dstb, wb, rows, acc):
        cid = lax.axis_index("c")
        sid = lax.axis_index("s")
        wid = cid * NS + sid
        pltpu.sync_copy(dst_hbm.at[wid], dstb)
        pltpu.sync_copy(w_hbm.at[wid], wb)

        zeros16 = jnp.zeros((16,), jnp.float32)

        @pl.loop(0, CHUNK)
        def _(r):
            rows[r, pl.ds(0, 16)] = zeros16

        @pl.loop(sid, NSLAB, step=NS)
        def _(s):
            pltpu.sync_copy(rows, acc.at[pl.ds(s * CHUNK, CHUNK)])

        plsc.subcore_barrier()

        ones16 = jnp.ones((16,), jnp.float32)

        @pl.loop(0, NSLAB)
        def _(g):
            @pl.loop(0, CHUNK, step=16)
            def _(i):
                wv = wb[g, pl.ds(i, 16)]
                for k in range(16):
                    rows[i + k, pl.ds(0, 16)] = ones16 * wv[k]

            pltpu.sync_copy(rows, acc.at[dstb.at[g]], add=True)

        plsc.subcore_barrier()

        @pl.loop(sid, NSLAB, step=NS)
        def _(s):
            pltpu.sync_copy(acc.at[pl.ds(s * CHUNK, CHUNK)],
                            out_hbm.at[cid, pl.ds(s * CHUNK, CHUNK)])

    return deg_kernel(dst_p3, w_p3)


def _agg_sc(src_p3, dst_p3, w_p3, h):
    CH = CHUNK
    NCH = EPT // CH
    PHASES = 2
    BCH = NCH // PHASES
    SLABS = NPAD // CH
    D = H1

    @functools.partial(
        pl.kernel,
        out_type=jax.ShapeDtypeStruct((NC, NPAD, D), jnp.float32),
        mesh=_mesh,
        compiler_params=_sc_params,
        scratch_types=[
            pltpu.VMEM((BCH, CH), jnp.int32),
            pltpu.VMEM((BCH, CH), jnp.int32),
            pltpu.VMEM((BCH, CH), jnp.float32),
            pltpu.VMEM((CH, D), jnp.float32),
            pltpu.VMEM((CH, D), jnp.float32),
            pltpu.VMEM_SHARED((NPAD, D), jnp.float32),
            pltpu.SemaphoreType.DMA,
            pltpu.SemaphoreType.DMA,
        ],
    )
    def agg_kernel(src_hbm, dst_hbm, w_hbm, h_hbm, out_hbm,
                   srcb, dstb, wb, rows0, rows1, acc, sem0, sem1):
        cid = lax.axis_index("c")
        sid = lax.axis_index("s")
        wid = cid * NS + sid

        zeros16 = jnp.zeros((16,), jnp.float32)

        @pl.loop(0, CH)
        def _(r):
            for j in range(D // 16):
                rows0[r, pl.ds(j * 16, 16)] = zeros16

        @pl.loop(sid, SLABS, step=NS)
        def _(s):
            pltpu.sync_copy(rows0, acc.at[pl.ds(s * CH, CH)])

        plsc.subcore_barrier()

        def start_gather(g, buf, sem):
            pltpu.make_async_copy(h_hbm.at[srcb.at[g]], buf, sem).start()

        def wait_gather(g, buf, sem):
            pltpu.make_async_copy(h_hbm.at[srcb.at[g]], buf, sem).wait()

        def process(gg, buf, sem):
            wait_gather(gg, buf, sem)

            @pl.loop(0, CH, step=16)
            def _(i):
                wv = wb[gg, pl.ds(i, 16)]
                for k in range(16):
                    ws = wv[k]
                    for j in range(D // 16):
                        sl = (i + k, pl.ds(j * 16, 16))
                        buf[sl] = buf[sl] * ws

            pltpu.sync_copy(buf, acc.at[dstb.at[gg]], add=True)

        for p in range(PHASES):
            pltpu.sync_copy(src_hbm.at[wid, pl.ds(p * BCH, BCH)], srcb)
            pltpu.sync_copy(dst_hbm.at[wid, pl.ds(p * BCH, BCH)], dstb)
            pltpu.sync_copy(w_hbm.at[wid, pl.ds(p * BCH, BCH)], wb)

            start_gather(0, rows0, sem0)
            start_gather(1, rows1, sem1)

            @pl.loop(0, BCH - 2, step=2)
            def _(g):
                for buf, sem, off in ((rows0, sem0, 0), (rows1, sem1, 1)):
                    gg = g + off
                    process(gg, buf, sem)
                    start_gather(gg + 2, buf, sem)

            process(BCH - 2, rows0, sem0)
            process(BCH - 1, rows1, sem1)

        plsc.subcore_barrier()

        @pl.loop(sid, SLABS, step=NS)
        def _(s):
            pltpu.sync_copy(acc.at[pl.ds(s * CH, CH)],
                            out_hbm.at[cid, pl.ds(s * CH, CH)])

    return agg_kernel(src_p3, dst_p3, w_p3, h)


def _tc1a(x, W1):
    def body(x_ref, w1_ref, h1_ref):
        h1_ref[...] = lax.dot_general(
            x_ref[...], w1_ref[...], (((1,), (1,)), ((), ())),
            preferred_element_type=jnp.float32)

    return pl.pallas_call(
        body,
        out_shape=jax.ShapeDtypeStruct((N, H1), jnp.float32),
    )(x, W1)


def _tc1b(h1, degp):
    def body(h1_ref, degp_ref, dinv_ref, h1s_ref):
        deg = 1.0 + degp_ref[0, :, 0:1] + degp_ref[1, :, 0:1]
        dinv = lax.rsqrt(deg)
        dinv_ref[...] = dinv
        h1s_ref[...] = h1_ref[...] * dinv[:N]

    return pl.pallas_call(
        body,
        out_shape=(jax.ShapeDtypeStruct((NPAD, 1), jnp.float32),
                   jax.ShapeDtypeStruct((N, H1), jnp.float32)),
    )(h1, degp)


def _tc2(p, h1s, dinv, b1, gamma1, beta1, W2):
    def body(p_ref, h1s_ref, dinv_ref, b1_ref, g1_ref, be1_ref, w2_ref,
             h2s_ref):
        dv = dinv_ref[pl.ds(0, N), :]
        agg = p_ref[0, :N, :] + p_ref[1, :N, :] + h1s_ref[...]
        out1 = dv * agg + b1_ref[...]
        mean = jnp.mean(out1, axis=0, keepdims=True)
        var = jnp.mean((out1 - mean) ** 2, axis=0, keepdims=True)
        hbn = (out1 - mean) / jnp.sqrt(var + EPS) * g1_ref[...] + be1_ref[...]
        hr = jnp.maximum(hbn, 0.0)
        h2 = lax.dot_general(hr, w2_ref[...], (((1,), (1,)), ((), ())),
                             preferred_element_type=jnp.float32)
        h2s = h2 * dv
        h2s_ref[...] = jnp.concatenate(
            [h2s, jnp.zeros((N, H1 - H2), jnp.float32)], axis=1)

    return pl.pallas_call(
        body,
        out_shape=jax.ShapeDtypeStruct((N, H1), jnp.float32),
    )(p, h1s, dinv, b1, gamma1, beta1, W2)


def _tc3(q, h2s, dinv, b2, gamma2, beta2, Wlin, blin):
    def body(q_ref, h2s_ref, dinv_ref, b2_ref, g2_ref, be2_ref, wl_ref,
             bl_ref, y_ref):
        dv = dinv_ref[pl.ds(0, N), :]
        agg = q_ref[0, :N, :H2] + q_ref[1, :N, :H2] + h2s_ref[:, :H2]
        out2 = dv * agg + b2_ref[...]
        mean = jnp.mean(out2, axis=0, keepdims=True)
        var = jnp.mean((out2 - mean) ** 2, axis=0, keepdims=True)
        hbn = (out2 - mean) / jnp.sqrt(var + EPS) * g2_ref[...] + be2_ref[...]
        hr = jnp.maximum(hbn, 0.0)
        y = lax.dot_general(hr, wl_ref[...], (((1,), (1,)), ((), ())),
                            preferred_element_type=jnp.float32)
        y_ref[...] = y + bl_ref[0, 0]

    return pl.pallas_call(
        body,
        out_shape=jax.ShapeDtypeStruct((N, H1), jnp.float32),
    )(q, h2s, dinv, b2, gamma2, beta2, Wlin, blin)


def kernel(x, edge_index, edge_weight, W1, b1, gamma1, beta1,
           W2, b2, gamma2, beta2, Wlin, blin):
    src = edge_index[0]
    dst = edge_index[1]
    pad = EP - E
    shp = (NW, EPT // CHUNK, CHUNK)
    pad_idx = jnp.arange(pad, dtype=jnp.int32) % N
    src_p3 = jnp.concatenate([src, pad_idx]).reshape(shp)
    dst_p3 = jnp.concatenate([dst, pad_idx]).reshape(shp)
    w_p3 = jnp.concatenate(
        [edge_weight, jnp.zeros((pad,), jnp.float32)]).reshape(shp)

    h1 = _tc1a(x, W1)
    degp = _deg_sc(dst_p3, w_p3)
    dinv, h1s = _tc1b(h1, degp)
    p1 = _agg_sc(src_p3, dst_p3, w_p3, h1s)
    h2s = _tc2(p1, h1s, dinv, b1.reshape(1, H1), gamma1.reshape(1, H1),
               beta1.reshape(1, H1), W2)
    q2 = _agg_sc(src_p3, dst_p3, w_p3, h2s)
    wl_b = jnp.broadcast_to(Wlin, (H1, H2))
    y = _tc3(q2, h2s, dinv, b2.reshape(1, H2), gamma2.reshape(1, H2),
             beta2.reshape(1, H2), wl_b, blin.reshape(1, 1))
    return y[:, 0]
